# baseline (speedup 1.0000x reference)
import functools

import jax
import jax.numpy as jnp
from jax.experimental import pallas as pl
from jax.experimental.pallas import tpu as pltpu


def _enc_block(x_ref, w_ref, b_ref, o_ref, *, K):
    xs = x_ref[...]
    BB, TT, _, S = xs.shape
    m = jnp.sum(xs, axis=2) * (1.0 / K)
    m2 = m.reshape(BB * TT, S)
    z = jnp.dot(m2, w_ref[...], preferred_element_type=jnp.float32)
    z = z + b_ref[...]
    zt = jnp.tile(z, (1, K))
    o_ref[...] = zt.reshape(BB, TT, K * z.shape[1])


def kernel(x, W, b):
    B, T, K, S = x.shape
    Z = W.shape[1]
    BB = 16
    out = pl.pallas_call(
        functools.partial(_enc_block, K=K),
        grid=(B // BB,),
        in_specs=[
            pl.BlockSpec((BB, T, K, S), lambda i: (i, 0, 0, 0)),
            pl.BlockSpec((S, Z), lambda i: (0, 0)),
            pl.BlockSpec((1, Z), lambda i: (0, 0)),
        ],
        out_specs=pl.BlockSpec((BB, T, K * Z), lambda i: (i, 0, 0)),
        out_shape=jax.ShapeDtypeStruct((B, T, K * Z), jnp.float32),
        compiler_params=pltpu.CompilerParams(
            dimension_semantics=("arbitrary",), vmem_limit_bytes=120*1024*1024),
    )(x, W, b.reshape(1, Z))
    return out.reshape(B, T, K, Z)
